# baseline (speedup 1.0000x reference)
import jax
import jax.numpy as jnp
from jax.experimental import pallas as pl
from jax.experimental.pallas import tpu as pltpu
from jax.experimental.pallas import tpu_sc as plsc

E = 8
F_IN = 128
H1 = 64
H2 = 64
F_OUT = 16

B = 256
GW = 256

def _vector_mesh():
    return plsc.VectorSubcoreMesh(core_axis_name="c", subcore_axis_name="s")


def _celu(x):
    return jnp.where(x > 0, x, jnp.exp(jnp.minimum(x, 0.0)) - 1.0)


def _routing(el, n_pad):
    n = el.shape[0]
    eye = (el[:, None] == jnp.arange(E, dtype=el.dtype)[None, :]).astype(jnp.int32)
    counts = jnp.sum(eye, axis=0)
    ranks = jnp.cumsum(eye, axis=0) - eye
    rank = jnp.sum(ranks * eye, axis=1)
    nblk_e = (counts + B - 1) // B
    pstart = (jnp.cumsum(nblk_e) - nblk_e) * B
    dest = jnp.sum(eye * pstart[None, :], axis=1) + rank
    sidx = jnp.full((n_pad,), n, jnp.int32).at[dest].set(jnp.arange(n, dtype=jnp.int32))
    gather_idx = jnp.minimum(sidx, n - 1)
    blk_start = jnp.arange(n_pad // B, dtype=jnp.int32) * B
    block_eid = jnp.sum((blk_start[:, None] >= pstart[None, 1:]).astype(jnp.int32), axis=1)
    return gather_idx, sidx, block_eid


def _sc_gather(x, idx, n_pad):
    @pl.kernel(out_type=jax.ShapeDtypeStruct((n_pad, F_IN), x.dtype),
               mesh=_vector_mesh(), scratch_types=[])
    def gk(x_hbm, i_hbm, o_hbm):
        def body(i_vmem, o_vmem):
            pltpu.sync_copy(x_hbm.at[i_vmem.at[0]], o_vmem)

        pltpu.emit_pipeline(
            body,
            grid=(n_pad // GW,),
            in_specs=[pl.BlockSpec((1, GW), lambda i: (0, i))],
            out_specs=[pl.BlockSpec((GW, F_IN), lambda i: (i, 0))],
            core_axis_name=("c", "s"),
            dimension_semantics=(pltpu.PARALLEL,),
        )(i_hbm, o_hbm)

    return gk(x, idx.reshape(1, n_pad))


def _sc_scatter(rows, idx, n_out, n_pad):
    @pl.kernel(out_type=jax.ShapeDtypeStruct((n_out, F_IN), rows.dtype),
               mesh=_vector_mesh(), scratch_types=[])
    def sk(r_hbm, i_hbm, o_hbm):
        def body(r_vmem, i_vmem):
            pltpu.sync_copy(r_vmem, o_hbm.at[i_vmem.at[0]])

        pltpu.emit_pipeline(
            body,
            grid=(n_pad // GW,),
            in_specs=[pl.BlockSpec((GW, F_IN), lambda i: (i, 0)),
                      pl.BlockSpec((1, GW), lambda i: (0, i))],
            out_specs=[],
            core_axis_name=("c", "s"),
            dimension_semantics=(pltpu.PARALLEL,),
        )(r_hbm, i_hbm)

    return sk(rows, idx.reshape(1, n_pad))


def _mlp_block_kernel(eid_ref, x_ref, w1_ref, b1_ref, w2_ref, b2_ref, w3_ref, b3_ref, o_ref):
    xb = x_ref[...].astype(jnp.bfloat16)
    z = jax.lax.dot_general(xb, w1_ref[0].astype(jnp.bfloat16),
                            (((1,), (1,)), ((), ())), preferred_element_type=jnp.float32)
    h = _celu(z + b1_ref[0]).astype(jnp.bfloat16)
    z = jax.lax.dot_general(h, w2_ref[0].astype(jnp.bfloat16),
                            (((1,), (1,)), ((), ())), preferred_element_type=jnp.float32)
    h = _celu(z + b2_ref[0]).astype(jnp.bfloat16)
    z = jax.lax.dot_general(h, w3_ref[0].astype(jnp.bfloat16),
                            (((1,), (1,)), ((), ())), preferred_element_type=jnp.float32)
    o_ref[:, :F_OUT] = z + b3_ref[0]
    o_ref[:, F_OUT:] = jnp.zeros((z.shape[0], F_IN - F_OUT), jnp.float32)


def _tc_mlp(xg, block_eid, W1, b1, W2, b2, W3, b3, n_pad):
    nblk = n_pad // B
    grid_spec = pltpu.PrefetchScalarGridSpec(
        num_scalar_prefetch=1,
        grid=(nblk,),
        in_specs=[
            pl.BlockSpec((B, F_IN), lambda i, eid: (i, 0)),
            pl.BlockSpec((1, H1, F_IN), lambda i, eid: (eid[i], 0, 0)),
            pl.BlockSpec((1, 1, H1), lambda i, eid: (eid[i], 0, 0)),
            pl.BlockSpec((1, H2, H1), lambda i, eid: (eid[i], 0, 0)),
            pl.BlockSpec((1, 1, H2), lambda i, eid: (eid[i], 0, 0)),
            pl.BlockSpec((1, F_OUT, H2), lambda i, eid: (eid[i], 0, 0)),
            pl.BlockSpec((1, 1, F_OUT), lambda i, eid: (eid[i], 0, 0)),
        ],
        out_specs=pl.BlockSpec((B, F_IN), lambda i, eid: (i, 0)),
    )
    return pl.pallas_call(
        _mlp_block_kernel,
        grid_spec=grid_spec,
        out_shape=jax.ShapeDtypeStruct((n_pad, F_IN), jnp.float32),
    )(block_eid, xg, W1, b1.reshape(E, 1, H1), W2, b2.reshape(E, 1, H2),
      W3, b3.reshape(E, 1, F_OUT))


def kernel(elements, features, W1, b1, W2, b2, W3, b3):
    n, M, f = features.shape
    N = n * M
    n_pad = N + E * B
    el = elements.reshape(N)
    x = features.reshape(N, f)

    gather_idx, scatter_idx, block_eid = _routing(el, n_pad)
    xg = _sc_gather(x, gather_idx, n_pad)
    og = _tc_mlp(xg, block_eid, W1, b1, W2, b2, W3, b3, n_pad)
    y_ext = _sc_scatter(og, scatter_idx, N + E, n_pad)
    return (elements, y_ext[:N, :F_OUT].reshape(n, M, F_OUT))


# speedup vs baseline: 4.5183x; 4.5183x over previous
import jax
import jax.numpy as jnp
from jax.experimental import pallas as pl

E = 8
F_IN = 128
H1 = 64
H2 = 64
F_OUT = 16
B = 1024


def _celu(x):
    return jnp.where(x > 0, x, jnp.exp(jnp.minimum(x, 0.0)) - 1.0)


def _replicate_masked(h, el):
    zero = jnp.zeros_like(h)
    return jnp.concatenate([jnp.where(el == e, h, zero) for e in range(E)], axis=1)


def _mlp_block_kernel(el_ref, x_ref, w1_ref, b1_ref, w2_ref, b2_ref, w3_ref, b3_ref, o_ref):
    el = el_ref[...]
    xb = x_ref[...].astype(jnp.bfloat16)
    onehot = (el == jax.lax.broadcasted_iota(jnp.int32, (el.shape[0], E), 1)
              ).astype(jnp.bfloat16)

    def dot(a, b):
        return jax.lax.dot_general(a, b, (((1,), (0,)), ((), ())),
                                   preferred_element_type=jnp.float32)

    z = dot(_replicate_masked(xb, el), w1_ref[...]) + dot(onehot, b1_ref[...])
    h = _celu(z).astype(jnp.bfloat16)
    z = dot(_replicate_masked(h, el), w2_ref[...]) + dot(onehot, b2_ref[...])
    h = _celu(z).astype(jnp.bfloat16)
    o_ref[...] = dot(_replicate_masked(h, el), w3_ref[...]) + dot(onehot, b3_ref[...])


def kernel(elements, features, W1, b1, W2, b2, W3, b3):
    n, M, f = features.shape
    N = n * M
    nblk = N // B
    x = features.reshape(N, f)
    el2 = elements.reshape(N, 1)

    W1s = W1.transpose(0, 2, 1).reshape(E * F_IN, H1).astype(jnp.bfloat16)
    W2s = W2.transpose(0, 2, 1).reshape(E * H1, H2).astype(jnp.bfloat16)
    W3s = W3.transpose(0, 2, 1).reshape(E * H2, F_OUT).astype(jnp.bfloat16)
    b1h = b1.astype(jnp.bfloat16)
    b2h = b2.astype(jnp.bfloat16)
    b3h = b3.astype(jnp.bfloat16)

    grid_spec = pl.GridSpec(
        grid=(nblk,),
        in_specs=[
            pl.BlockSpec((B, 1), lambda i: (i, 0)),
            pl.BlockSpec((B, F_IN), lambda i: (i, 0)),
            pl.BlockSpec((E * F_IN, H1), lambda i: (0, 0)),
            pl.BlockSpec((E, H1), lambda i: (0, 0)),
            pl.BlockSpec((E * H1, H2), lambda i: (0, 0)),
            pl.BlockSpec((E, H2), lambda i: (0, 0)),
            pl.BlockSpec((E * H2, F_OUT), lambda i: (0, 0)),
            pl.BlockSpec((E, F_OUT), lambda i: (0, 0)),
        ],
        out_specs=pl.BlockSpec((B, F_OUT), lambda i: (i, 0)),
    )
    y = pl.pallas_call(
        _mlp_block_kernel,
        grid_spec=grid_spec,
        out_shape=jax.ShapeDtypeStruct((N, F_OUT), jnp.float32),
    )(el2, x, W1s, b1h, W2s, b2h, W3s, b3h)
    return (elements, y.reshape(n, M, F_OUT))


# speedup vs baseline: 8.1246x; 1.7982x over previous
import jax
import jax.numpy as jnp
from jax.experimental import pallas as pl

E = 8
F_IN = 128
H1 = 64
H2 = 64
F_OUT = 16
B = 1024


def _celu(x):
    return jnp.where(x > 0, x, jnp.exp(jnp.minimum(x, 0.0)) - 1.0)


def _mlp_block_kernel(el_ref, x_ref, w1_ref, b1_ref, w2_ref, b2_ref, w3_ref, b3_ref, o_ref):
    el = el_ref[0]
    xT = jnp.transpose(x_ref[...], (1, 0)).astype(jnp.bfloat16)
    masks = [el == e for e in range(E)]

    def layer(hT, w_ref, b_ref, width):
        z = jnp.zeros((width, hT.shape[1]), dtype=jnp.float32)
        for e in range(E):
            we = w_ref[e].astype(jnp.bfloat16)
            ze = jax.lax.dot_general(we, hT, (((1,), (0,)), ((), ())),
                                     preferred_element_type=jnp.float32)
            z = jnp.where(masks[e], ze + b_ref[e], z)
        return z

    h = _celu(layer(xT, w1_ref, b1_ref, H1)).astype(jnp.bfloat16)
    h = _celu(layer(h, w2_ref, b2_ref, H2)).astype(jnp.bfloat16)
    o_ref[...] = layer(h, w3_ref, b3_ref, F_OUT)


def kernel(elements, features, W1, b1, W2, b2, W3, b3):
    n, M, f = features.shape
    N = n * M
    nblk = N // B
    x = features.reshape(N, f)
    el3 = elements.reshape(nblk, 1, B)

    grid_spec = pl.GridSpec(
        grid=(nblk,),
        in_specs=[
            pl.BlockSpec((1, 1, B), lambda i: (i, 0, 0)),
            pl.BlockSpec((B, F_IN), lambda i: (i, 0)),
            pl.BlockSpec((E, H1, F_IN), lambda i: (0, 0, 0)),
            pl.BlockSpec((E, H1, 1), lambda i: (0, 0, 0)),
            pl.BlockSpec((E, H2, H1), lambda i: (0, 0, 0)),
            pl.BlockSpec((E, H2, 1), lambda i: (0, 0, 0)),
            pl.BlockSpec((E, F_OUT, H2), lambda i: (0, 0, 0)),
            pl.BlockSpec((E, F_OUT, 1), lambda i: (0, 0, 0)),
        ],
        out_specs=pl.BlockSpec((F_OUT, B), lambda i: (0, i)),
    )
    yT = pl.pallas_call(
        _mlp_block_kernel,
        grid_spec=grid_spec,
        out_shape=jax.ShapeDtypeStruct((F_OUT, N), jnp.float32),
    )(el3, x, W1, b1.reshape(E, H1, 1), W2, b2.reshape(E, H2, 1),
      W3, b3.reshape(E, F_OUT, 1))
    return (elements, yT.T.reshape(n, M, F_OUT))
